# Initial kernel scaffold; baseline (speedup 1.0000x reference)
import jax
import jax.numpy as jnp
from jax.experimental import pallas as pl


def kernel(x, edge_index, W1, b1, g1, be1, W2, b2, g2, be2, Wd, bd):
    raise NotImplementedError("write your pallas kernel here")


# speedup vs baseline: 26.4541x; 26.4541x over previous
import functools

import jax
import jax.numpy as jnp
from jax import lax
from jax.experimental import pallas as pl
from jax.experimental.pallas import tpu as pltpu
from jax.experimental.pallas import tpu_sc as plsc

EPS = 1e-5
NC = 2
NS = 16
NW = NC * NS
CH = 80


def _sc_mesh():
    return plsc.VectorSubcoreMesh(
        core_axis_name="c", subcore_axis_name="s", num_cores=NC,
        num_subcores=NS)


_SC_PARAMS = pltpu.CompilerParams(needs_layout_passes=False,
                                  use_tc_tiling_on_sc=False)


def _deg_body(dst_hbm, out_hbm, idx_v, deg_v):
    e = dst_hbm.shape[0]
    n = deg_v.shape[0]
    et = e // NW
    wid = lax.axis_index("c") * NS + lax.axis_index("s")
    pltpu.sync_copy(dst_hbm.at[pl.ds(wid * et, et)], idx_v)
    zeros16 = jnp.zeros((16,), jnp.float32)

    def zb(i, _):
        deg_v[pl.ds(i * 16, 16)] = zeros16
        return ()
    lax.fori_loop(0, n // 16, zb, ())

    ones16 = jnp.ones((16,), jnp.float32)

    def sb(i, _):
        idx = idx_v[pl.ds(i * 16, 16)]
        plsc.addupdate_scatter(deg_v, [idx], ones16)
        return ()
    lax.fori_loop(0, et // 16, sb, ())
    pltpu.sync_copy(deg_v, out_hbm.at[pl.ds(wid * n, n)])


def _deg_partials(dst, n):
    e = dst.shape[0]
    return pl.kernel(
        _deg_body,
        out_type=jax.ShapeDtypeStruct((NW * n,), jnp.float32),
        mesh=_sc_mesh(),
        compiler_params=_SC_PARAMS,
        scratch_types=[
            pltpu.VMEM((e // NW,), jnp.int32),
            pltpu.VMEM((n,), jnp.float32),
        ],
    )(dst).reshape(NW, n)


def _agg1_body(src_hbm, dst_hbm, hs_hbm, out_hbm, idxs_v, idxd_v, rows_v,
               zbuf_v, acc_sh, sem):
    e = src_hbm.shape[0]
    npad, d = hs_hbm.shape
    et = e // NW
    nch = et // CH
    sl = npad // NS
    zr = zbuf_v.shape[0]
    c = lax.axis_index("c")
    s = lax.axis_index("s")
    wid = c * NS + s
    base = wid * et

    zeros16 = jnp.zeros((16,), jnp.float32)

    def zb(i, _):
        zbuf_v[i // (d // 16), pl.ds((i % (d // 16)) * 16, 16)] = zeros16
        return ()
    lax.fori_loop(0, zr * (d // 16), zb, ())

    def zc(i, _):
        pltpu.sync_copy(zbuf_v, acc_sh.at[pl.ds(s * sl + i * zr, zr)])
        return ()
    lax.fori_loop(0, sl // zr, zc, ())
    plsc.subcore_barrier()

    def cb(j, _):
        pltpu.sync_copy(src_hbm.at[pl.ds(base + j * CH, CH)], idxs_v)
        pltpu.sync_copy(dst_hbm.at[pl.ds(base + j * CH, CH)], idxd_v)
        pltpu.async_copy(hs_hbm.at[idxs_v], rows_v, sem).wait()
        pltpu.sync_copy(rows_v, acc_sh.at[idxd_v], add=True)
        return ()
    lax.fori_loop(0, nch, cb, ())
    plsc.subcore_barrier()

    def wb(i, _):
        pltpu.sync_copy(acc_sh.at[pl.ds(s * sl + i * zr, zr)], zbuf_v)
        pltpu.sync_copy(zbuf_v, out_hbm.at[c, pl.ds(s * sl + i * zr, zr)])
        return ()
    lax.fori_loop(0, sl // zr, wb, ())


def _agg1(src, dst, hs_pad):
    npad, d = hs_pad.shape
    return pl.kernel(
        _agg1_body,
        out_type=jax.ShapeDtypeStruct((NC, npad, d), jnp.float32),
        mesh=_sc_mesh(),
        compiler_params=_SC_PARAMS,
        scratch_types=[
            pltpu.VMEM((CH,), jnp.int32),
            pltpu.VMEM((CH,), jnp.int32),
            pltpu.VMEM((CH, d), jnp.float32),
            pltpu.VMEM((128, d), jnp.float32),
            pltpu.VMEM_SHARED((npad, d), jnp.float32),
            pltpu.SemaphoreType.DMA,
        ],
    )(src, dst, hs_pad)


def _agg2_body(src_hbm, dst_hbm, tst_hbm, out_hbm, ts0_v, ts1_v, acc0_v,
               acc1_v, idxs_v, idxd_v):
    e = src_hbm.shape[0]
    n = ts0_v.shape[0]
    et = e // NW
    wid = lax.axis_index("c") * NS + lax.axis_index("s")
    base = wid * et
    pltpu.sync_copy(tst_hbm.at[0], ts0_v)
    pltpu.sync_copy(tst_hbm.at[1], ts1_v)
    pltpu.sync_copy(src_hbm.at[pl.ds(base, et)], idxs_v)
    pltpu.sync_copy(dst_hbm.at[pl.ds(base, et)], idxd_v)
    zeros16 = jnp.zeros((16,), jnp.float32)

    def zb(i, _):
        acc0_v[pl.ds(i * 16, 16)] = zeros16
        acc1_v[pl.ds(i * 16, 16)] = zeros16
        return ()
    lax.fori_loop(0, n // 16, zb, ())

    def sb(i, _):
        si = idxs_v[pl.ds(i * 16, 16)]
        di = idxd_v[pl.ds(i * 16, 16)]
        v0 = plsc.load_gather(ts0_v, [si])
        v1 = plsc.load_gather(ts1_v, [si])
        plsc.addupdate_scatter(acc0_v, [di], v0)
        plsc.addupdate_scatter(acc1_v, [di], v1)
        return ()
    lax.fori_loop(0, et // 16, sb, ())
    pltpu.sync_copy(acc0_v, out_hbm.at[pl.ds((wid * 2) * n, n)])
    pltpu.sync_copy(acc1_v, out_hbm.at[pl.ds((wid * 2 + 1) * n, n)])


def _agg2(src, dst, tst):
    e = src.shape[0]
    n = tst.shape[1]
    return pl.kernel(
        _agg2_body,
        out_type=jax.ShapeDtypeStruct((NW * 2 * n,), jnp.float32),
        mesh=_sc_mesh(),
        compiler_params=_SC_PARAMS,
        scratch_types=[
            pltpu.VMEM((n,), jnp.float32),
            pltpu.VMEM((n,), jnp.float32),
            pltpu.VMEM((n,), jnp.float32),
            pltpu.VMEM((n,), jnp.float32),
            pltpu.VMEM((e // NW,), jnp.int32),
            pltpu.VMEM((e // NW,), jnp.int32),
        ],
    )(src, dst, tst).reshape(NW, 2, n)


def _tc_b_body(x_ref, w1_ref, degt_ref, hs_ref, dinv_ref):
    n = x_ref.shape[0]
    npad, d = hs_ref.shape
    deg = jnp.sum(degt_ref[...], axis=1, keepdims=True) + 1.0
    dinv = lax.rsqrt(deg)
    h0 = jnp.dot(x_ref[...], w1_ref[...], preferred_element_type=jnp.float32)
    hs_ref[:n] = h0 * dinv
    hs_ref[n:] = jnp.zeros((npad - n, d), jnp.float32)
    dinv_ref[...] = dinv


def _tc_d_body(p_ref, hs_ref, dinv_ref, b1_ref, g1_ref, be1_ref, w2_ref,
               ts_ref):
    dinv = dinv_ref[...]
    n = dinv.shape[0]
    p = (p_ref[0] + p_ref[1])[:n]
    pre = (p + hs_ref[:n]) * dinv + b1_ref[...][None, :]
    mu = jnp.mean(pre, axis=0, keepdims=True)
    var = jnp.mean((pre - mu) ** 2, axis=0, keepdims=True)
    h1 = jnp.maximum(
        (pre - mu) * lax.rsqrt(var + EPS) * g1_ref[...][None, :]
        + be1_ref[...][None, :], 0.0)
    t = jnp.dot(h1, w2_ref[...], preferred_element_type=jnp.float32)
    ts_ref[...] = t * dinv


def _tc_f_body(q_ref, ts_ref, dinv_ref, b2_ref, g2_ref, be2_ref, wd_ref,
               bd_ref, out_ref):
    q = jnp.sum(q_ref[...], axis=0)
    z = (q.T + ts_ref[...]) * dinv_ref[...] + b2_ref[...][None, :]
    mu = jnp.mean(z, axis=0, keepdims=True)
    var = jnp.mean((z - mu) ** 2, axis=0, keepdims=True)
    zbn = ((z - mu) * lax.rsqrt(var + EPS) * g2_ref[...][None, :]
           + be2_ref[...][None, :])
    out_ref[...] = (jnp.dot(zbn, wd_ref[...],
                            preferred_element_type=jnp.float32)
                    + bd_ref[...][None, :])


def kernel(x, edge_index, W1, b1, g1, be1, W2, b2, g2, be2, Wd, bd):
    n, d_in = x.shape
    hid = W1.shape[1]
    src = edge_index[0]
    dst = edge_index[1]

    degp = _deg_partials(dst, n)

    npad = ((n + NS * 128 - 1) // (NS * 128)) * (NS * 128)
    hs, dinv = pl.pallas_call(
        _tc_b_body,
        out_shape=(jax.ShapeDtypeStruct((npad, hid), jnp.float32),
                   jax.ShapeDtypeStruct((n, 1), jnp.float32)),
    )(x, W1, degp.T)

    P = _agg1(src, dst, hs)

    ts = pl.pallas_call(
        _tc_d_body,
        out_shape=jax.ShapeDtypeStruct((n, W2.shape[1]), jnp.float32),
    )(P, hs, dinv, b1, g1, be1, W2)

    Q = _agg2(src, dst, ts.T)

    out = pl.pallas_call(
        _tc_f_body,
        out_shape=jax.ShapeDtypeStruct((n, Wd.shape[1]), jnp.float32),
    )(Q, ts, dinv, b2, g2, be2, Wd, bd)
    return out


# speedup vs baseline: 56.1961x; 2.1243x over previous
import functools

import jax
import jax.numpy as jnp
from jax import lax
from jax.experimental import pallas as pl
from jax.experimental.pallas import tpu as pltpu
from jax.experimental.pallas import tpu_sc as plsc

EPS = 1e-5
NC = 2
NS = 16
NW = NC * NS
CH = 125


def _sc_mesh():
    return plsc.VectorSubcoreMesh(
        core_axis_name="c", subcore_axis_name="s", num_cores=NC,
        num_subcores=NS)


_SC_PARAMS = pltpu.CompilerParams(needs_layout_passes=False,
                                  use_tc_tiling_on_sc=False)


def _deg_body(dst_hbm, out_hbm, idx_v, deg_v):
    e = dst_hbm.shape[0]
    n = deg_v.shape[0]
    et = e // NW
    wid = lax.axis_index("c") * NS + lax.axis_index("s")
    pltpu.sync_copy(dst_hbm.at[pl.ds(wid * et, et)], idx_v)
    zeros16 = jnp.zeros((16,), jnp.float32)

    def zb(i, _):
        deg_v[pl.ds(i * 16, 16)] = zeros16
        return ()
    lax.fori_loop(0, n // 16, zb, ())

    ones16 = jnp.ones((16,), jnp.float32)

    def sb(i, _):
        idx = idx_v[pl.ds(i * 16, 16)]
        plsc.addupdate_scatter(deg_v, [idx], ones16)
        return ()
    lax.fori_loop(0, et // 16, sb, ())
    pltpu.sync_copy(deg_v, out_hbm.at[pl.ds(wid * n, n)])


def _deg_partials(dst, n):
    e = dst.shape[0]
    return pl.kernel(
        _deg_body,
        out_type=jax.ShapeDtypeStruct((NW * n,), jnp.float32),
        mesh=_sc_mesh(),
        compiler_params=_SC_PARAMS,
        scratch_types=[
            pltpu.VMEM((e // NW,), jnp.int32),
            pltpu.VMEM((n,), jnp.float32),
        ],
    )(dst).reshape(NW, n)


NBUF = 4


def _agg1_body(src_hbm, dst_hbm, hs_hbm, out_hbm, idxs_v, idxd_v, rows_v,
               zbuf_v, acc_sh, sem0, sem1, sem2, sem3):
    nrow, ch = src_hbm.shape
    npad, d = hs_hbm.shape
    nch = nrow // NW
    sl = npad // NS
    zr = zbuf_v.shape[0]
    c = lax.axis_index("c")
    s = lax.axis_index("s")
    wid = c * NS + s
    sems = [sem0, sem1, sem2, sem3]

    pltpu.sync_copy(src_hbm.at[pl.ds(wid * nch, nch)], idxs_v)
    pltpu.sync_copy(dst_hbm.at[pl.ds(wid * nch, nch)], idxd_v)

    zeros16 = jnp.zeros((16,), jnp.float32)

    def zb(i, _):
        zbuf_v[i // (d // 16), pl.ds((i % (d // 16)) * 16, 16)] = zeros16
        return ()
    lax.fori_loop(0, zr * (d // 16), zb, ())

    def zc(i, _):
        pltpu.sync_copy(zbuf_v, acc_sh.at[pl.ds(s * sl + i * zr, zr)])
        return ()
    lax.fori_loop(0, sl // zr, zc, ())
    plsc.subcore_barrier()

    ngrp = nch // NBUF
    for b in range(NBUF):
        pltpu.async_copy(hs_hbm.at[idxs_v.at[b]], rows_v.at[b], sems[b])

    def cb(g, _):
        for b in range(NBUF):
            j = g * NBUF + b
            pltpu.make_async_copy(hs_hbm.at[idxs_v.at[j]], rows_v.at[b],
                                  sems[b]).wait()
            pltpu.sync_copy(rows_v.at[b], acc_sh.at[idxd_v.at[j]], add=True)

            @pl.when(g < ngrp - 1)
            def _():
                pltpu.async_copy(hs_hbm.at[idxs_v.at[j + NBUF]],
                                 rows_v.at[b], sems[b])
        return ()
    lax.fori_loop(0, ngrp, cb, ())
    plsc.subcore_barrier()

    def wb(i, _):
        pltpu.sync_copy(acc_sh.at[pl.ds(s * sl + i * zr, zr)], zbuf_v)
        pltpu.sync_copy(zbuf_v, out_hbm.at[c, pl.ds(s * sl + i * zr, zr)])
        return ()
    lax.fori_loop(0, sl // zr, wb, ())


def _agg1(src2, dst2, hs_pad):
    npad, d = hs_pad.shape
    nrow, ch = src2.shape
    nch = nrow // NW
    return pl.kernel(
        _agg1_body,
        out_type=jax.ShapeDtypeStruct((NC, npad, d), jnp.float32),
        mesh=_sc_mesh(),
        compiler_params=_SC_PARAMS,
        scratch_types=[
            pltpu.VMEM((nch, ch), jnp.int32),
            pltpu.VMEM((nch, ch), jnp.int32),
            pltpu.VMEM((NBUF, ch, d), jnp.float32),
            pltpu.VMEM((128, d), jnp.float32),
            pltpu.VMEM_SHARED((npad, d), jnp.float32),
            pltpu.SemaphoreType.DMA,
            pltpu.SemaphoreType.DMA,
            pltpu.SemaphoreType.DMA,
            pltpu.SemaphoreType.DMA,
        ],
    )(src2, dst2, hs_pad)


def _agg2_body(src_hbm, dst_hbm, tst_hbm, out_hbm, ts0_v, ts1_v, acc0_v,
               acc1_v, idxs_v, idxd_v):
    e = src_hbm.shape[0]
    n = ts0_v.shape[0]
    et = e // NW
    wid = lax.axis_index("c") * NS + lax.axis_index("s")
    base = wid * et
    pltpu.sync_copy(tst_hbm.at[0], ts0_v)
    pltpu.sync_copy(tst_hbm.at[1], ts1_v)
    pltpu.sync_copy(src_hbm.at[pl.ds(base, et)], idxs_v)
    pltpu.sync_copy(dst_hbm.at[pl.ds(base, et)], idxd_v)
    zeros16 = jnp.zeros((16,), jnp.float32)

    def zb(i, _):
        acc0_v[pl.ds(i * 16, 16)] = zeros16
        acc1_v[pl.ds(i * 16, 16)] = zeros16
        return ()
    lax.fori_loop(0, n // 16, zb, ())

    def sb(i, _):
        si = idxs_v[pl.ds(i * 16, 16)]
        di = idxd_v[pl.ds(i * 16, 16)]
        v0 = plsc.load_gather(ts0_v, [si])
        v1 = plsc.load_gather(ts1_v, [si])
        plsc.addupdate_scatter(acc0_v, [di], v0)
        plsc.addupdate_scatter(acc1_v, [di], v1)
        return ()
    lax.fori_loop(0, et // 16, sb, ())
    pltpu.sync_copy(acc0_v, out_hbm.at[pl.ds((wid * 2) * n, n)])
    pltpu.sync_copy(acc1_v, out_hbm.at[pl.ds((wid * 2 + 1) * n, n)])


def _agg2(src, dst, tst):
    e = src.shape[0]
    n = tst.shape[1]
    return pl.kernel(
        _agg2_body,
        out_type=jax.ShapeDtypeStruct((NW * 2 * n,), jnp.float32),
        mesh=_sc_mesh(),
        compiler_params=_SC_PARAMS,
        scratch_types=[
            pltpu.VMEM((n,), jnp.float32),
            pltpu.VMEM((n,), jnp.float32),
            pltpu.VMEM((n,), jnp.float32),
            pltpu.VMEM((n,), jnp.float32),
            pltpu.VMEM((e // NW,), jnp.int32),
            pltpu.VMEM((e // NW,), jnp.int32),
        ],
    )(src, dst, tst).reshape(NW, 2, n)


def _tc_b_body(x_ref, w1_ref, degt_ref, hs_ref, dinv_ref):
    n = x_ref.shape[0]
    npad, d = hs_ref.shape
    deg = jnp.sum(degt_ref[...], axis=1, keepdims=True) + 1.0
    dinv = lax.rsqrt(deg)
    h0 = jnp.dot(x_ref[...], w1_ref[...], preferred_element_type=jnp.float32)
    hs_ref[:n] = h0 * dinv
    hs_ref[n:] = jnp.zeros((npad - n, d), jnp.float32)
    dinv_ref[...] = dinv


def _tc_d_body(p_ref, hs_ref, dinv_ref, b1_ref, g1_ref, be1_ref, w2_ref,
               ts_ref):
    dinv = dinv_ref[...]
    n = dinv.shape[0]
    p = (p_ref[0] + p_ref[1])[:n]
    pre = (p + hs_ref[:n]) * dinv + b1_ref[...][None, :]
    mu = jnp.mean(pre, axis=0, keepdims=True)
    var = jnp.mean((pre - mu) ** 2, axis=0, keepdims=True)
    h1 = jnp.maximum(
        (pre - mu) * lax.rsqrt(var + EPS) * g1_ref[...][None, :]
        + be1_ref[...][None, :], 0.0)
    t = jnp.dot(h1, w2_ref[...], preferred_element_type=jnp.float32)
    ts_ref[...] = t * dinv


def _tc_f_body(q_ref, ts_ref, dinv_ref, b2_ref, g2_ref, be2_ref, wd_ref,
               bd_ref, out_ref):
    q = jnp.sum(q_ref[...], axis=0)
    z = (q.T + ts_ref[...]) * dinv_ref[...] + b2_ref[...][None, :]
    mu = jnp.mean(z, axis=0, keepdims=True)
    var = jnp.mean((z - mu) ** 2, axis=0, keepdims=True)
    zbn = ((z - mu) * lax.rsqrt(var + EPS) * g2_ref[...][None, :]
           + be2_ref[...][None, :])
    out_ref[...] = (jnp.dot(zbn, wd_ref[...],
                            preferred_element_type=jnp.float32)
                    + bd_ref[...][None, :])


def kernel(x, edge_index, W1, b1, g1, be1, W2, b2, g2, be2, Wd, bd):
    n, d_in = x.shape
    hid = W1.shape[1]
    src = edge_index[0]
    dst = edge_index[1]

    degp = _deg_partials(dst, n)

    npad = ((n + NS * 128 - 1) // (NS * 128)) * (NS * 128)
    hs, dinv = pl.pallas_call(
        _tc_b_body,
        out_shape=(jax.ShapeDtypeStruct((npad, hid), jnp.float32),
                   jax.ShapeDtypeStruct((n, 1), jnp.float32)),
    )(x, W1, degp.T)

    P = _agg1(src.reshape(-1, CH), dst.reshape(-1, CH), hs)

    ts = pl.pallas_call(
        _tc_d_body,
        out_shape=jax.ShapeDtypeStruct((n, W2.shape[1]), jnp.float32),
    )(P, hs, dinv, b1, g1, be1, W2)

    Q = _agg2(src, dst, ts.T)

    out = pl.pallas_call(
        _tc_f_body,
        out_shape=jax.ShapeDtypeStruct((n, Wd.shape[1]), jnp.float32),
    )(Q, ts, dinv, b2, g2, be2, Wd, bd)
    return out


# speedup vs baseline: 64.3468x; 1.1450x over previous
import functools

import jax
import jax.numpy as jnp
from jax import lax
from jax.experimental import pallas as pl
from jax.experimental.pallas import tpu as pltpu
from jax.experimental.pallas import tpu_sc as plsc

EPS = 1e-5
NC = 2
NS = 16
NW = NC * NS
CH = 125


def _sc_mesh():
    return plsc.VectorSubcoreMesh(
        core_axis_name="c", subcore_axis_name="s", num_cores=NC,
        num_subcores=NS)


_SC_PARAMS = pltpu.CompilerParams(needs_layout_passes=False,
                                  use_tc_tiling_on_sc=False)


def _deg_body(ei_hbm, out_hbm, idx_v, deg_v, sem):
    e = ei_hbm.shape[1]
    n = deg_v.shape[0]
    et = e // NW
    wid = lax.axis_index("c") * NS + lax.axis_index("s")
    cp = pltpu.async_copy(ei_hbm.at[1, pl.ds(wid * et, et)], idx_v, sem)
    zeros16 = jnp.zeros((16,), jnp.float32)

    def zb(i, _):
        deg_v[pl.ds(i * 16, 16)] = zeros16
        return ()
    lax.fori_loop(0, n // 16, zb, ())
    cp.wait()

    ones16 = jnp.ones((16,), jnp.float32)

    def sb(i, _):
        idx = idx_v[pl.ds(i * 16, 16)]
        plsc.addupdate_scatter(deg_v, [idx], ones16)
        return ()
    lax.fori_loop(0, et // 16, sb, ())
    pltpu.sync_copy(deg_v, out_hbm.at[pl.ds(wid * n, n)])


def _deg_partials(ei, n):
    e = ei.shape[1]
    return pl.kernel(
        _deg_body,
        out_type=jax.ShapeDtypeStruct((NW * n,), jnp.float32),
        mesh=_sc_mesh(),
        compiler_params=_SC_PARAMS,
        scratch_types=[
            pltpu.VMEM((e // NW,), jnp.int32),
            pltpu.VMEM((n,), jnp.float32),
            pltpu.SemaphoreType.DMA,
        ],
    )(ei).reshape(NW, n)


NBUF = 4


def _agg1_body(ei_hbm, hs_hbm, out_hbm, idxs_v, idxd_v, rows_v,
               zbuf_v, acc_sh, *sems):
    _, nrow, ch = ei_hbm.shape
    npad, d = hs_hbm.shape
    nch = nrow // NW
    sl = npad // NS
    zr = zbuf_v.shape[0]
    c = lax.axis_index("c")
    s = lax.axis_index("s")
    wid = c * NS + s

    cp_s = pltpu.async_copy(ei_hbm.at[0, pl.ds(wid * nch, nch)], idxs_v,
                            sems[NBUF])
    cp_d = pltpu.async_copy(ei_hbm.at[1, pl.ds(wid * nch, nch)], idxd_v,
                            sems[NBUF + 1])

    zeros16 = jnp.zeros((16,), jnp.float32)

    def zb(i, _):
        zbuf_v[i // (d // 16), pl.ds((i % (d // 16)) * 16, 16)] = zeros16
        return ()
    lax.fori_loop(0, zr * (d // 16), zb, ())

    def zc(i, _):
        pltpu.sync_copy(zbuf_v, acc_sh.at[pl.ds(s * sl + i * zr, zr)])
        return ()
    lax.fori_loop(0, sl // zr, zc, ())
    cp_s.wait()
    cp_d.wait()
    plsc.subcore_barrier()

    ngrp = nch // NBUF
    for b in range(NBUF):
        pltpu.async_copy(hs_hbm.at[idxs_v.at[b]], rows_v.at[b], sems[b])

    def cb(g, _):
        for b in range(NBUF):
            j = g * NBUF + b
            pltpu.make_async_copy(hs_hbm.at[idxs_v.at[j]], rows_v.at[b],
                                  sems[b]).wait()
            pltpu.sync_copy(rows_v.at[b], acc_sh.at[idxd_v.at[j]], add=True)

            @pl.when(g < ngrp - 1)
            def _():
                pltpu.async_copy(hs_hbm.at[idxs_v.at[j + NBUF]],
                                 rows_v.at[b], sems[b])
        return ()
    lax.fori_loop(0, ngrp, cb, ())
    plsc.subcore_barrier()

    def wb(i, _):
        pltpu.sync_copy(acc_sh.at[pl.ds(s * sl + i * zr, zr)], zbuf_v)
        pltpu.sync_copy(zbuf_v, out_hbm.at[c, pl.ds(s * sl + i * zr, zr)])
        return ()
    lax.fori_loop(0, sl // zr, wb, ())


def _agg1(eiR, hs_pad):
    npad, d = hs_pad.shape
    _, nrow, ch = eiR.shape
    nch = nrow // NW
    return pl.kernel(
        _agg1_body,
        out_type=jax.ShapeDtypeStruct((NC, npad, d), jnp.float32),
        mesh=_sc_mesh(),
        compiler_params=_SC_PARAMS,
        scratch_types=[
            pltpu.VMEM((nch, ch), jnp.int32),
            pltpu.VMEM((nch, ch), jnp.int32),
            pltpu.VMEM((NBUF, ch, d), jnp.float32),
            pltpu.VMEM((128, d), jnp.float32),
            pltpu.VMEM_SHARED((npad, d), jnp.float32),
        ] + [pltpu.SemaphoreType.DMA] * (NBUF + 2),
    )(eiR, hs_pad)


def _agg2_body(ei_hbm, tst_hbm, out_hbm, ts0_v, ts1_v, acc0_v,
               acc1_v, idxs_v, idxd_v, sem0, sem1, sem2, sem3):
    e = ei_hbm.shape[1]
    n = ts0_v.shape[0]
    et = e // NW
    wid = lax.axis_index("c") * NS + lax.axis_index("s")
    base = wid * et
    cp0 = pltpu.async_copy(tst_hbm.at[0], ts0_v, sem0)
    cp1 = pltpu.async_copy(tst_hbm.at[1], ts1_v, sem1)
    cp2 = pltpu.async_copy(ei_hbm.at[0, pl.ds(base, et)], idxs_v, sem2)
    cp3 = pltpu.async_copy(ei_hbm.at[1, pl.ds(base, et)], idxd_v, sem3)
    zeros16 = jnp.zeros((16,), jnp.float32)

    def zb(i, _):
        acc0_v[pl.ds(i * 16, 16)] = zeros16
        acc1_v[pl.ds(i * 16, 16)] = zeros16
        return ()
    lax.fori_loop(0, n // 16, zb, ())
    cp0.wait()
    cp1.wait()
    cp2.wait()
    cp3.wait()

    def sb(i, _):
        si = idxs_v[pl.ds(i * 16, 16)]
        di = idxd_v[pl.ds(i * 16, 16)]
        v0 = plsc.load_gather(ts0_v, [si])
        v1 = plsc.load_gather(ts1_v, [si])
        plsc.addupdate_scatter(acc0_v, [di], v0)
        plsc.addupdate_scatter(acc1_v, [di], v1)
        return ()
    lax.fori_loop(0, et // 16, sb, ())
    pltpu.sync_copy(acc0_v, out_hbm.at[pl.ds((wid * 2) * n, n)])
    pltpu.sync_copy(acc1_v, out_hbm.at[pl.ds((wid * 2 + 1) * n, n)])


def _agg2(ei, tst):
    e = ei.shape[1]
    n = tst.shape[1]
    return pl.kernel(
        _agg2_body,
        out_type=jax.ShapeDtypeStruct((NW * 2 * n,), jnp.float32),
        mesh=_sc_mesh(),
        compiler_params=_SC_PARAMS,
        scratch_types=[
            pltpu.VMEM((n,), jnp.float32),
            pltpu.VMEM((n,), jnp.float32),
            pltpu.VMEM((n,), jnp.float32),
            pltpu.VMEM((n,), jnp.float32),
            pltpu.VMEM((e // NW,), jnp.int32),
            pltpu.VMEM((e // NW,), jnp.int32),
            pltpu.SemaphoreType.DMA,
            pltpu.SemaphoreType.DMA,
            pltpu.SemaphoreType.DMA,
            pltpu.SemaphoreType.DMA,
        ],
    )(ei, tst).reshape(NW, 2, n)


def _tc_b_body(x_ref, w1_ref, degp_ref, hs_ref, dinv_ref):
    n = x_ref.shape[0]
    npad, d = hs_ref.shape
    deg = jnp.sum(degp_ref[...], axis=0)[:, None] + 1.0
    dinv = lax.rsqrt(deg)
    h0 = jnp.dot(x_ref[...], w1_ref[...], preferred_element_type=jnp.float32)
    hs_ref[:n] = h0 * dinv
    hs_ref[n:] = jnp.zeros((npad - n, d), jnp.float32)
    dinv_ref[...] = dinv


def _tc_d_body(p_ref, hs_ref, dinv_ref, b1_ref, g1_ref, be1_ref, w2_ref,
               ts_ref, tst_ref):
    dinv = dinv_ref[...]
    n = dinv.shape[0]
    p = (p_ref[0] + p_ref[1])[:n]
    pre = (p + hs_ref[:n]) * dinv + b1_ref[...][None, :]
    mu = jnp.mean(pre, axis=0, keepdims=True)
    var = jnp.mean((pre - mu) ** 2, axis=0, keepdims=True)
    h1 = jnp.maximum(
        (pre - mu) * lax.rsqrt(var + EPS) * g1_ref[...][None, :]
        + be1_ref[...][None, :], 0.0)
    t = jnp.dot(h1, w2_ref[...], preferred_element_type=jnp.float32)
    ts = t * dinv
    ts_ref[...] = ts
    tst_ref[...] = ts.T


def _tc_f_body(q_ref, ts_ref, dinv_ref, b2_ref, g2_ref, be2_ref, wd_ref,
               bd_ref, out_ref):
    q = jnp.sum(q_ref[...], axis=0)
    z = (q.T + ts_ref[...]) * dinv_ref[...] + b2_ref[...][None, :]
    mu = jnp.mean(z, axis=0, keepdims=True)
    var = jnp.mean((z - mu) ** 2, axis=0, keepdims=True)
    zbn = ((z - mu) * lax.rsqrt(var + EPS) * g2_ref[...][None, :]
           + be2_ref[...][None, :])
    out_ref[...] = (jnp.dot(zbn, wd_ref[...],
                            preferred_element_type=jnp.float32)
                    + bd_ref[...][None, :])


def kernel(x, edge_index, W1, b1, g1, be1, W2, b2, g2, be2, Wd, bd):
    n, d_in = x.shape
    hid = W1.shape[1]

    degp = _deg_partials(edge_index, n)

    npad = ((n + NS * 128 - 1) // (NS * 128)) * (NS * 128)
    hs, dinv = pl.pallas_call(
        _tc_b_body,
        out_shape=(jax.ShapeDtypeStruct((npad, hid), jnp.float32),
                   jax.ShapeDtypeStruct((n, 1), jnp.float32)),
    )(x, W1, degp)

    P = _agg1(edge_index.reshape(2, -1, CH), hs)

    ts, tst = pl.pallas_call(
        _tc_d_body,
        out_shape=(jax.ShapeDtypeStruct((n, W2.shape[1]), jnp.float32),
                   jax.ShapeDtypeStruct((W2.shape[1], n), jnp.float32)),
    )(P, hs, dinv, b1, g1, be1, W2)

    Q = _agg2(edge_index, tst)

    out = pl.pallas_call(
        _tc_f_body,
        out_shape=jax.ShapeDtypeStruct((n, Wd.shape[1]), jnp.float32),
    )(Q, ts, dinv, b2, g2, be2, Wd, bd)
    return out
